# baseline (speedup 1.0000x reference)
import functools

import jax
import jax.numpy as jnp
from jax import lax
from jax.experimental import pallas as pl
from jax.experimental.pallas import tpu as pltpu
from jax.experimental.pallas import tpu_sc as plsc

_NUM_CORES = 2
_NUM_SUBCORES = 16
_NW = _NUM_CORES * _NUM_SUBCORES

_D = 128
_V0, _V1, _V2 = 5, 6, 2
_VT = _V0 * _V1 * _V2

_CHUNK = 80
_NBUF = 8


def _build_table_kernel(w0_ref, w1_ref, w2_ref, t_ref):
    for i in range(_V0):
        for j in range(_V1):
            base = w0_ref[pl.ds(i, 1), :] + w1_ref[pl.ds(j, 1), :]
            for k in range(_V2):
                r = (i * _V1 + j) * _V2 + k
                t_ref[pl.ds(r, 1), :] = base + w2_ref[pl.ds(k, 1), :]


def _build_table(w0, w1, w2):
    return pl.pallas_call(
        _build_table_kernel,
        out_shape=jax.ShapeDtypeStruct((_VT, _D), jnp.float32),
    )(w0, w1, w2)


def _make_lookup(n_edges):
    assert n_edges % (_NW * _CHUNK) == 0
    epw = n_edges // _NW
    nchunk = epw // _CHUNK
    assert nchunk >= 2 * _NBUF
    r_full = (nchunk - _NBUF) // _NBUF
    tail_start = r_full * _NBUF
    mesh = plsc.VectorSubcoreMesh(
        core_axis_name="c",
        subcore_axis_name="s",
        num_cores=_NUM_CORES,
        num_subcores=_NUM_SUBCORES,
    )

    @functools.partial(
        pl.kernel,
        mesh=mesh,
        out_type=jax.ShapeDtypeStruct((n_edges, _D), jnp.float32),
        scratch_types=[
            pltpu.VMEM((epw,), jnp.int32),
            pltpu.VMEM((epw,), jnp.int32),
            pltpu.VMEM((epw,), jnp.int32),
            pltpu.VMEM((nchunk, _CHUNK), jnp.int32),
            pltpu.VMEM((_NBUF * _CHUNK, _D), jnp.float32),
            pltpu.VMEM_SHARED((_VT, _D), jnp.float32),
            [pltpu.SemaphoreType.DMA] * _NBUF,
            [pltpu.SemaphoreType.DMA] * _NBUF,
        ],
    )
    def lookup(a0_hbm, a1_hbm, a2_hbm, t_hbm, out_hbm,
               a0_v, a1_v, a2_v, idx_v, rows_v, t_sh, gsem, ssem):
        cid = lax.axis_index("c")
        sid = lax.axis_index("s")
        wid = sid * _NUM_CORES + cid
        base = wid * epw

        @pl.when(sid == 0)
        def _():
            pltpu.sync_copy(t_hbm, t_sh)

        pltpu.sync_copy(a0_hbm.at[pl.ds(base, epw)], a0_v)
        pltpu.sync_copy(a1_hbm.at[pl.ds(base, epw)], a1_v)
        pltpu.sync_copy(a2_hbm.at[pl.ds(base, epw)], a2_v)

        def cbody(g, carry):
            for t in range(_CHUNK // 16):
                s = g * _CHUNK + t * 16
                a0 = a0_v[pl.ds(s, 16)]
                a1 = a1_v[pl.ds(s, 16)]
                a2 = a2_v[pl.ds(s, 16)]
                idx_v[g, pl.ds(t * 16, 16)] = (a0 * _V1 + a1) * _V2 + a2
            return carry

        lax.fori_loop(0, nchunk, cbody, 0)
        plsc.subcore_barrier()

        def buf(b, n=1):
            return rows_v.at[pl.ds(b * _CHUNK, n * _CHUNK), :]

        def issue_gather(g, b):
            pltpu.async_copy(t_sh.at[idx_v.at[g]], buf(b), gsem[b])

        def wait_gather(g, b):
            pltpu.make_async_copy(t_sh.at[idx_v.at[g]], buf(b), gsem[b]).wait()

        def out_slice(g, n=1):
            return out_hbm.at[pl.ds(base + g * _CHUNK, n * _CHUNK), :]

        def issue_scatter(g, b, n=1):
            pltpu.async_copy(buf(b, n), out_slice(g, n), ssem[b])

        def wait_scatter(g, b, n=1):
            pltpu.make_async_copy(buf(b, n), out_slice(g, n), ssem[b]).wait()

        for b in range(_NBUF):
            issue_gather(b, b)

        def rbody(r, carry):
            g0 = r * _NBUF
            for q in range(2):
                for b in range(q * 4, q * 4 + 4):
                    wait_gather(g0 + b, b)
                issue_scatter(g0 + q * 4, q * 4, n=4)
            for q in range(2):
                wait_scatter(g0 + q * 4, q * 4, n=4)
                for b in range(q * 4, q * 4 + 4):
                    issue_gather(g0 + b + _NBUF, b)
            return carry

        lax.fori_loop(0, r_full, rbody, 0)

        for g in range(tail_start, nchunk):
            b = g % _NBUF
            wait_gather(g, b)
            issue_scatter(g, b)
            if g + _NBUF < nchunk:
                wait_scatter(g, b)
                issue_gather(g + _NBUF, b)
        for g in range(nchunk - _NBUF, nchunk):
            wait_scatter(g, g % _NBUF)

    return lookup


@jax.jit
def kernel(edge_attr, W0, W1, W2):
    edge_attr = jnp.asarray(edge_attr, jnp.int32)
    table = _build_table(
        jnp.asarray(W0, jnp.float32),
        jnp.asarray(W1, jnp.float32),
        jnp.asarray(W2, jnp.float32),
    )
    lookup = _make_lookup(edge_attr.shape[0])
    return lookup(edge_attr[:, 0], edge_attr[:, 1], edge_attr[:, 2], table)


# speedup vs baseline: 1.0331x; 1.0331x over previous
import functools

import jax
import jax.numpy as jnp
from jax import lax
from jax.experimental import pallas as pl
from jax.experimental.pallas import tpu as pltpu
from jax.experimental.pallas import tpu_sc as plsc

_NUM_CORES = 2
_NUM_SUBCORES = 16
_NW = _NUM_CORES * _NUM_SUBCORES

_D = 128
_V0, _V1, _V2 = 5, 6, 2
_VT = _V0 * _V1 * _V2

_CHUNK = 80
_NBUF = 8


def _build_table_kernel(w0_ref, w1_ref, w2_ref, t_ref):
    for i in range(_V0):
        for j in range(_V1):
            base = w0_ref[pl.ds(i, 1), :] + w1_ref[pl.ds(j, 1), :]
            for k in range(_V2):
                r = (i * _V1 + j) * _V2 + k
                t_ref[pl.ds(r, 1), :] = base + w2_ref[pl.ds(k, 1), :]


def _build_table(w0, w1, w2):
    return pl.pallas_call(
        _build_table_kernel,
        out_shape=jax.ShapeDtypeStruct((_VT, _D), jnp.float32),
    )(w0, w1, w2)


def _make_lookup(n_edges):
    assert n_edges % (_NW * _CHUNK) == 0
    epw = n_edges // _NW
    nchunk = epw // _CHUNK
    assert nchunk >= 2 * _NBUF
    r_full = (nchunk - _NBUF) // _NBUF
    tail_start = r_full * _NBUF
    mesh = plsc.VectorSubcoreMesh(
        core_axis_name="c",
        subcore_axis_name="s",
        num_cores=_NUM_CORES,
        num_subcores=_NUM_SUBCORES,
    )

    @functools.partial(
        pl.kernel,
        mesh=mesh,
        out_type=jax.ShapeDtypeStruct((n_edges, _D), jnp.float32),
        scratch_types=[
            pltpu.VMEM((epw,), jnp.int32),
            pltpu.VMEM((epw,), jnp.int32),
            pltpu.VMEM((epw,), jnp.int32),
            pltpu.VMEM((nchunk, _CHUNK), jnp.int32),
            pltpu.VMEM((_NBUF, _CHUNK, _D), jnp.float32),
            pltpu.VMEM_SHARED((_VT, _D), jnp.float32),
            [pltpu.SemaphoreType.DMA] * _NBUF,
            [pltpu.SemaphoreType.DMA] * _NBUF,
        ],
    )
    def lookup(a0_hbm, a1_hbm, a2_hbm, t_hbm, out_hbm,
               a0_v, a1_v, a2_v, idx_v, rows_v, t_sh, gsem, ssem):
        cid = lax.axis_index("c")
        sid = lax.axis_index("s")
        wid = sid * _NUM_CORES + cid
        base = wid * epw

        @pl.when(sid == 0)
        def _():
            pltpu.sync_copy(t_hbm, t_sh)

        pltpu.sync_copy(a0_hbm.at[pl.ds(base, epw)], a0_v)
        pltpu.sync_copy(a1_hbm.at[pl.ds(base, epw)], a1_v)
        pltpu.sync_copy(a2_hbm.at[pl.ds(base, epw)], a2_v)

        def compute_idx(g):
            for t in range(_CHUNK // 16):
                s = g * _CHUNK + t * 16
                a0 = a0_v[pl.ds(s, 16)]
                a1 = a1_v[pl.ds(s, 16)]
                a2 = a2_v[pl.ds(s, 16)]
                idx_v[g, pl.ds(t * 16, 16)] = (a0 * _V1 + a1) * _V2 + a2

        def cbody(g, carry):
            compute_idx(g)
            return carry

        lax.fori_loop(0, _NBUF, cbody, 0)
        plsc.subcore_barrier()

        def issue_gather(g, b):
            pltpu.async_copy(t_sh.at[idx_v.at[g]], rows_v.at[b], gsem[b])

        def wait_gather(g, b):
            pltpu.make_async_copy(t_sh.at[idx_v.at[g]], rows_v.at[b], gsem[b]).wait()

        def out_slice(g):
            return out_hbm.at[pl.ds(base + g * _CHUNK, _CHUNK), :]

        def issue_scatter(g, b):
            pltpu.async_copy(rows_v.at[b], out_slice(g), ssem[b])

        def wait_scatter(g, b):
            pltpu.make_async_copy(rows_v.at[b], out_slice(g), ssem[b]).wait()

        for b in range(_NBUF):
            issue_gather(b, b)

        def rbody(r, carry):
            g0 = r * _NBUF
            for b in range(_NBUF):
                compute_idx(g0 + b + _NBUF)
                wait_gather(g0 + b, b)
                issue_scatter(g0 + b, b)
            for b in range(_NBUF):
                wait_scatter(g0 + b, b)
                issue_gather(g0 + b + _NBUF, b)
            return carry

        lax.fori_loop(0, r_full, rbody, 0)

        for g in range(tail_start, nchunk):
            b = g % _NBUF
            if g + _NBUF < nchunk:
                compute_idx(g + _NBUF)
            wait_gather(g, b)
            issue_scatter(g, b)
            if g + _NBUF < nchunk:
                wait_scatter(g, b)
                issue_gather(g + _NBUF, b)
        for g in range(nchunk - _NBUF, nchunk):
            wait_scatter(g, g % _NBUF)

    return lookup


@jax.jit
def kernel(edge_attr, W0, W1, W2):
    edge_attr = jnp.asarray(edge_attr, jnp.int32)
    table = _build_table(
        jnp.asarray(W0, jnp.float32),
        jnp.asarray(W1, jnp.float32),
        jnp.asarray(W2, jnp.float32),
    )
    lookup = _make_lookup(edge_attr.shape[0])
    return lookup(edge_attr[:, 0], edge_attr[:, 1], edge_attr[:, 2], table)


# speedup vs baseline: 1.0469x; 1.0134x over previous
import functools

import jax
import jax.numpy as jnp
from jax import lax
from jax.experimental import pallas as pl
from jax.experimental.pallas import tpu as pltpu
from jax.experimental.pallas import tpu_sc as plsc

_NUM_CORES = 2
_NUM_SUBCORES = 16
_NW = _NUM_CORES * _NUM_SUBCORES

_D = 128
_V0, _V1, _V2 = 5, 6, 2
_VT = _V0 * _V1 * _V2

_CHUNK = 80
_NBUF = 8


def _build_table_kernel(w0_ref, w1_ref, w2_ref, t_ref):
    for i in range(_V0):
        for j in range(_V1):
            base = w0_ref[pl.ds(i, 1), :] + w1_ref[pl.ds(j, 1), :]
            for k in range(_V2):
                r = (i * _V1 + j) * _V2 + k
                t_ref[pl.ds(r, 1), :] = base + w2_ref[pl.ds(k, 1), :]


def _build_table(w0, w1, w2):
    return pl.pallas_call(
        _build_table_kernel,
        out_shape=jax.ShapeDtypeStruct((_VT, _D), jnp.float32),
    )(w0, w1, w2)


def _make_lookup(n_edges):
    assert n_edges % (_NW * _CHUNK) == 0
    epw = n_edges // _NW
    nchunk = epw // _CHUNK
    assert nchunk >= 2 * _NBUF
    r_full = (nchunk - _NBUF) // _NBUF
    tail_start = r_full * _NBUF
    mesh = plsc.VectorSubcoreMesh(
        core_axis_name="c",
        subcore_axis_name="s",
        num_cores=_NUM_CORES,
        num_subcores=_NUM_SUBCORES,
    )

    @functools.partial(
        pl.kernel,
        mesh=mesh,
        out_type=jax.ShapeDtypeStruct((n_edges, _D), jnp.float32),
        scratch_types=[
            pltpu.VMEM((epw,), jnp.int32),
            pltpu.VMEM((epw,), jnp.int32),
            pltpu.VMEM((epw,), jnp.int32),
            pltpu.VMEM((nchunk, _CHUNK), jnp.int32),
            pltpu.VMEM((_NBUF, _CHUNK, _D), jnp.float32),
            pltpu.VMEM_SHARED((_VT, _D), jnp.float32),
            [pltpu.SemaphoreType.DMA] * _NBUF,
            [pltpu.SemaphoreType.DMA] * _NBUF,
        ],
    )
    def lookup(a0_hbm, a1_hbm, a2_hbm, t_hbm, out_hbm,
               a0_v, a1_v, a2_v, idx_v, rows_v, t_sh, gsem, ssem):
        cid = lax.axis_index("c")
        sid = lax.axis_index("s")
        wid = sid * _NUM_CORES + cid
        base = wid * epw

        @pl.when(sid == 0)
        def _():
            pltpu.sync_copy(t_hbm, t_sh)

        d0 = pltpu.async_copy(a0_hbm.at[pl.ds(base, epw)], a0_v, ssem[0])
        d1 = pltpu.async_copy(a1_hbm.at[pl.ds(base, epw)], a1_v, ssem[1])
        d2 = pltpu.async_copy(a2_hbm.at[pl.ds(base, epw)], a2_v, ssem[2])
        d0.wait()
        d1.wait()
        d2.wait()

        def compute_idx(g):
            for t in range(_CHUNK // 16):
                s = g * _CHUNK + t * 16
                a0 = a0_v[pl.ds(s, 16)]
                a1 = a1_v[pl.ds(s, 16)]
                a2 = a2_v[pl.ds(s, 16)]
                idx_v[g, pl.ds(t * 16, 16)] = (a0 * _V1 + a1) * _V2 + a2

        def cbody(g, carry):
            compute_idx(g)
            return carry

        lax.fori_loop(0, _NBUF, cbody, 0)
        plsc.subcore_barrier()

        def issue_gather(g, b):
            pltpu.async_copy(t_sh.at[idx_v.at[g]], rows_v.at[b], gsem[b])

        def wait_gather(g, b):
            pltpu.make_async_copy(t_sh.at[idx_v.at[g]], rows_v.at[b], gsem[b]).wait()

        def out_slice(g):
            return out_hbm.at[pl.ds(base + g * _CHUNK, _CHUNK), :]

        def issue_scatter(g, b):
            pltpu.async_copy(rows_v.at[b], out_slice(g), ssem[b])

        def wait_scatter(g, b):
            pltpu.make_async_copy(rows_v.at[b], out_slice(g), ssem[b]).wait()

        for b in range(_NBUF):
            issue_gather(b, b)

        def rbody(r, carry):
            g0 = r * _NBUF
            for b in range(_NBUF):
                compute_idx(g0 + b + _NBUF)
                wait_gather(g0 + b, b)
                issue_scatter(g0 + b, b)
            for b in range(_NBUF):
                wait_scatter(g0 + b, b)
                issue_gather(g0 + b + _NBUF, b)
            return carry

        lax.fori_loop(0, r_full, rbody, 0)

        for g in range(tail_start, nchunk):
            b = g % _NBUF
            if g + _NBUF < nchunk:
                compute_idx(g + _NBUF)
            wait_gather(g, b)
            issue_scatter(g, b)
            if g + _NBUF < nchunk:
                wait_scatter(g, b)
                issue_gather(g + _NBUF, b)
        for g in range(nchunk - _NBUF, nchunk):
            wait_scatter(g, g % _NBUF)

    return lookup


@jax.jit
def kernel(edge_attr, W0, W1, W2):
    edge_attr = jnp.asarray(edge_attr, jnp.int32)
    table = _build_table(
        jnp.asarray(W0, jnp.float32),
        jnp.asarray(W1, jnp.float32),
        jnp.asarray(W2, jnp.float32),
    )
    lookup = _make_lookup(edge_attr.shape[0])
    return lookup(edge_attr[:, 0], edge_attr[:, 1], edge_attr[:, 2], table)
